# baseline (speedup 1.0000x reference)
import functools

import jax
import jax.numpy as jnp
from jax import lax
from jax.experimental import pallas as pl
from jax.experimental.pallas import tpu as pltpu
from jax.experimental.pallas import tpu_sc as plsc

N = 10000
E = 320000
D = 128

NC, NS = 2, 16
NW = NC * NS
CHUNK = 112
NCHUNK = 96
EPW = NCHUNK * CHUNK
SCH = 24
NSTAGE = NCHUNK // SCH
NBUF = 3
E_PAD = NW * EPW
NP = 10240
ACC_ROWS = 10112
ZR = ACC_ROWS // NS

_mesh = plsc.VectorSubcoreMesh(core_axis_name="c", subcore_axis_name="s")


def _zero_acc(acc, zer, s):
    pltpu.sync_copy(zer, acc.at[pl.ds(s * ZR, ZR)])


def _copy_out(acc, o0, o1, c, s):
    @pl.when(c == 0)
    def _():
        pltpu.sync_copy(acc.at[pl.ds(s * ZR, ZR)], o0.at[pl.ds(s * ZR, ZR)])
    @pl.when(c == 1)
    def _():
        pltpu.sync_copy(acc.at[pl.ds(s * ZR, ZR)], o1.at[pl.ds(s * ZR, ZR)])


def _deg_body(dst2d, ones_h, zer, o0, o1, didx, ones, acc):
    c = lax.axis_index("c")
    s = lax.axis_index("s")
    wid = s * NC + c
    _zero_acc(acc, zer, s)
    pltpu.sync_copy(ones_h, ones)
    plsc.subcore_barrier()
    def body(i, carry):
        pltpu.sync_copy(dst2d.at[pl.ds(wid * NCHUNK + i, 1)], didx)
        pltpu.sync_copy(ones, acc.at[didx.at[0]], add=True)
        return carry
    lax.fori_loop(0, NCHUNK, body, 0)
    plsc.subcore_barrier()
    _copy_out(acc, o0, o1, c, s)


_deg_call = pl.kernel(
    _deg_body,
    out_type=(jax.ShapeDtypeStruct((NP, D), jnp.float32),
              jax.ShapeDtypeStruct((NP, D), jnp.float32)),
    mesh=_mesh,
    scratch_types=[
        pltpu.VMEM((1, CHUNK), jnp.int32),
        pltpu.VMEM((CHUNK, D), jnp.float32),
        pltpu.VMEM_SHARED((ACC_ROWS, D), jnp.float32),
    ],
)


def _segsum_body(g, src2d, dst2d, zer, o0, o1, sidx, didx, rows0, rows1,
                 rows2, acc, sem0, sem1, sem2):
    c = lax.axis_index("c")
    s = lax.axis_index("s")
    wid = s * NC + c
    base = wid * NCHUNK
    _zero_acc(acc, zer, s)
    plsc.subcore_barrier()
    rows = (rows0, rows1, rows2)
    sems = (sem0, sem1, sem2)
    def stage(st, carry):
        sbase = base + st * SCH
        pltpu.sync_copy(src2d.at[pl.ds(sbase, SCH)], sidx)
        pltpu.sync_copy(dst2d.at[pl.ds(sbase, SCH)], didx)
        for b in range(NBUF):
            pltpu.async_copy(g.at[sidx.at[b]], rows[b], sems[b])
        def body(gi, c2):
            for b in range(NBUF):
                i = gi * NBUF + b
                pltpu.make_async_copy(g.at[sidx.at[i]], rows[b], sems[b]).wait()
                pltpu.sync_copy(rows[b], acc.at[didx.at[i]], add=True)
                @pl.when(i + NBUF < SCH)
                def _():
                    pltpu.async_copy(g.at[sidx.at[i + NBUF]], rows[b], sems[b])
            return c2
        lax.fori_loop(0, SCH // NBUF, body, 0)
        return carry
    lax.fori_loop(0, NSTAGE, stage, 0)
    plsc.subcore_barrier()
    _copy_out(acc, o0, o1, c, s)


_segsum_call = pl.kernel(
    _segsum_body,
    out_type=(jax.ShapeDtypeStruct((NP, D), jnp.float32),
              jax.ShapeDtypeStruct((NP, D), jnp.float32)),
    mesh=_mesh,
    scratch_types=[
        pltpu.VMEM((SCH, CHUNK), jnp.int32),
        pltpu.VMEM((SCH, CHUNK), jnp.int32),
        pltpu.VMEM((CHUNK, D), jnp.float32),
        pltpu.VMEM((CHUNK, D), jnp.float32),
        pltpu.VMEM((CHUNK, D), jnp.float32),
        pltpu.VMEM_SHARED((ACC_ROWS, D), jnp.float32),
        pltpu.SemaphoreType.DMA,
        pltpu.SemaphoreType.DMA,
        pltpu.SemaphoreType.DMA,
    ],
)


def _gather2_body(p, q, src2d, dst2d, pa, qa, sidx, didx, pr0, pr1, pr2,
                  qr0, qr1, qr2, psem0, psem1, psem2, qsem0, qsem1, qsem2):
    c = lax.axis_index("c")
    s = lax.axis_index("s")
    wid = s * NC + c
    base = wid * NCHUNK
    prows = (pr0, pr1, pr2)
    qrows = (qr0, qr1, qr2)
    psems = (psem0, psem1, psem2)
    qsems = (qsem0, qsem1, qsem2)
    def stage(st, carry):
        sbase = base + st * SCH
        pltpu.sync_copy(src2d.at[pl.ds(sbase, SCH)], sidx)
        pltpu.sync_copy(dst2d.at[pl.ds(sbase, SCH)], didx)
        for b in range(NBUF):
            pltpu.async_copy(p.at[sidx.at[b]], prows[b], psems[b])
            pltpu.async_copy(q.at[didx.at[b]], qrows[b], qsems[b])
        def body(gi, c2):
            for b in range(NBUF):
                i = gi * NBUF + b
                off = (sbase + i) * CHUNK
                pltpu.make_async_copy(p.at[sidx.at[i]], prows[b], psems[b]).wait()
                pltpu.sync_copy(prows[b], pa.at[pl.ds(off, CHUNK)])
                pltpu.make_async_copy(q.at[didx.at[i]], qrows[b], qsems[b]).wait()
                pltpu.sync_copy(qrows[b], qa.at[pl.ds(off, CHUNK)])
                @pl.when(i + NBUF < SCH)
                def _():
                    pltpu.async_copy(p.at[sidx.at[i + NBUF]], prows[b], psems[b])
                    pltpu.async_copy(q.at[didx.at[i + NBUF]], qrows[b], qsems[b])
            return c2
        lax.fori_loop(0, SCH // NBUF, body, 0)
        return carry
    lax.fori_loop(0, NSTAGE, stage, 0)


_gather2_call = pl.kernel(
    _gather2_body,
    out_type=(jax.ShapeDtypeStruct((E_PAD, D), jnp.float32),
              jax.ShapeDtypeStruct((E_PAD, D), jnp.float32)),
    mesh=_mesh,
    scratch_types=[
        pltpu.VMEM((SCH, CHUNK), jnp.int32),
        pltpu.VMEM((SCH, CHUNK), jnp.int32),
    ] + [pltpu.VMEM((CHUNK, D), jnp.float32)] * 6
      + [pltpu.SemaphoreType.DMA] * 6,
)


BN = 640
BE = 1000

_w_spec = pl.BlockSpec((D, D), lambda i: (0, 0))
_b_spec = pl.BlockSpec((1, D), lambda i: (0, 0))
_n_spec = pl.BlockSpec((BN, D), lambda i: (i, 0))
_d_spec = pl.BlockSpec((BN, 1), lambda i: (i, 0))
_e_spec = pl.BlockSpec((BE, D), lambda i: (i, 0))


def _dis_kern(d0, d1, o):
    o[...] = lax.rsqrt(d0[:, 0:1] + d1[:, 0:1] + 1.0)


def _pre_kern(x, w, d, o):
    o[...] = jnp.dot(x[...], w[...], preferred_element_type=jnp.float32) * d[...]


def _mid_kern(a0, a1, g, d, b, w, o):
    dis = d[...]
    h = jnp.maximum(dis * (a0[...] + a1[...] + g[...]) + b[...], 0.0)
    o[...] = jnp.dot(h, w[...], preferred_element_type=jnp.float32) * dis


def _final_kern(a0, a1, g, d, b3, wt, wb, bm1, po, qo):
    h3 = d[...] * (a0[...] + a1[...] + g[...]) + b3[...]
    po[...] = jnp.dot(h3, wt[...], preferred_element_type=jnp.float32) + bm1[...]
    qo[...] = jnp.dot(h3, wb[...], preferred_element_type=jnp.float32)


def _edge_kern(pa, qa, w, b, o):
    h = jnp.maximum(pa[...] + qa[...], 0.0)
    o[...] = jnp.dot(h, w[...], preferred_element_type=jnp.float32) + b[...]


_dis_call = pl.pallas_call(
    _dis_kern,
    grid=(NP // BN,),
    in_specs=[_n_spec, _n_spec],
    out_specs=_d_spec,
    out_shape=jax.ShapeDtypeStruct((NP, 1), jnp.float32),
)

_pre_call = pl.pallas_call(
    _pre_kern,
    grid=(NP // BN,),
    in_specs=[_n_spec, _w_spec, _d_spec],
    out_specs=_n_spec,
    out_shape=jax.ShapeDtypeStruct((NP, D), jnp.float32),
)

_mid_call = pl.pallas_call(
    _mid_kern,
    grid=(NP // BN,),
    in_specs=[_n_spec, _n_spec, _n_spec, _d_spec, _b_spec, _w_spec],
    out_specs=_n_spec,
    out_shape=jax.ShapeDtypeStruct((NP, D), jnp.float32),
)

_final_call = pl.pallas_call(
    _final_kern,
    grid=(NP // BN,),
    in_specs=[_n_spec, _n_spec, _n_spec, _d_spec, _b_spec, _w_spec, _w_spec,
              _b_spec],
    out_specs=(_n_spec, _n_spec),
    out_shape=(jax.ShapeDtypeStruct((NP, D), jnp.float32),
               jax.ShapeDtypeStruct((NP, D), jnp.float32)),
)

_edge_call = pl.pallas_call(
    _edge_kern,
    grid=(E // BE,),
    in_specs=[_e_spec, _e_spec, _w_spec, _b_spec],
    out_specs=_e_spec,
    out_shape=jax.ShapeDtypeStruct((E, D), jnp.float32),
)


def kernel(x, edge_index, W1, b1, W2, b2, W3, b3, Wm1, bm1, Wm2, bm2):
    pad = E_PAD - E
    src2d = jnp.concatenate(
        [edge_index[0], jnp.zeros((pad,), jnp.int32)]).reshape(-1, CHUNK)
    dst2d = jnp.concatenate(
        [edge_index[1], jnp.full((pad,), N, jnp.int32)]).reshape(-1, CHUNK)
    b1r, b2r, b3r = b1.reshape(1, D), b2.reshape(1, D), b3.reshape(1, D)
    bm1r, bm2r = bm1.reshape(1, D), bm2.reshape(1, D)
    wt, wb = Wm1[:D], Wm1[D:]
    xp = jnp.concatenate([x, jnp.zeros((NP - N, D), jnp.float32)])

    zer = jnp.zeros((ZR, D), jnp.float32)
    d0, d1 = _deg_call(dst2d, jnp.ones((CHUNK, D), jnp.float32), zer)
    dis = _dis_call(d0, d1)
    g1 = _pre_call(xp, W1, dis)
    a0, a1 = _segsum_call(g1, src2d, dst2d, zer)
    g2 = _mid_call(a0, a1, g1, dis, b1r, W2)
    a0, a1 = _segsum_call(g2, src2d, dst2d, zer)
    g3 = _mid_call(a0, a1, g2, dis, b2r, W3)
    a0, a1 = _segsum_call(g3, src2d, dst2d, zer)
    p, q = _final_call(a0, a1, g3, dis, b3r, wt, wb, bm1r)
    pa, qa = _gather2_call(p, q, src2d, dst2d)
    return _edge_call(pa, qa, Wm2, bm2r)


# speedup vs baseline: 1.8835x; 1.8835x over previous
import functools

import jax
import jax.numpy as jnp
from jax import lax
from jax.experimental import pallas as pl
from jax.experimental.pallas import tpu as pltpu
from jax.experimental.pallas import tpu_sc as plsc

N = 10000
E = 320000
D = 128

NC, NS = 2, 16
NW = NC * NS
CHUNK = 128
NCHUNK = 80
EPW = NCHUNK * CHUNK
SCH = 16
NSTAGE = NCHUNK // SCH
NBUF = 2
E_PAD = NW * EPW
NP = 10240
ACC_ROWS = NP
ZR = ACC_ROWS // NS

_mesh = plsc.VectorSubcoreMesh(core_axis_name="c", subcore_axis_name="s")


def _zero_acc(acc, zer, s):
    pltpu.sync_copy(zer, acc.at[pl.ds(s * ZR, ZR)])


def _copy_out(acc, o0, o1, c, s):
    @pl.when(c == 0)
    def _():
        pltpu.sync_copy(acc.at[pl.ds(s * ZR, ZR)], o0.at[pl.ds(s * ZR, ZR)])
    @pl.when(c == 1)
    def _():
        pltpu.sync_copy(acc.at[pl.ds(s * ZR, ZR)], o1.at[pl.ds(s * ZR, ZR)])


def _deg_body(dst2d, ones_h, zer, o0, o1, didx, ones, acc):
    c = lax.axis_index("c")
    s = lax.axis_index("s")
    wid = s * NC + c
    _zero_acc(acc, zer, s)
    pltpu.sync_copy(ones_h, ones)
    plsc.subcore_barrier()
    def body(i, carry):
        pltpu.sync_copy(dst2d.at[pl.ds(wid * NCHUNK + i, 1)], didx)
        pltpu.sync_copy(ones, acc.at[didx.at[0]], add=True)
        return carry
    lax.fori_loop(0, NCHUNK, body, 0)
    plsc.subcore_barrier()
    _copy_out(acc, o0, o1, c, s)


_deg_call = pl.kernel(
    _deg_body,
    out_type=(jax.ShapeDtypeStruct((NP, D), jnp.float32),
              jax.ShapeDtypeStruct((NP, D), jnp.float32)),
    mesh=_mesh,
    scratch_types=[
        pltpu.VMEM((1, CHUNK), jnp.int32),
        pltpu.VMEM((CHUNK, D), jnp.float32),
        pltpu.VMEM_SHARED((ACC_ROWS, D), jnp.float32),
    ],
)


def _segsum_body(g, src2d, dst2d, zer, o0, o1, sidx, didx, rows0, rows1,
                 acc, sem0, sem1):
    c = lax.axis_index("c")
    s = lax.axis_index("s")
    wid = s * NC + c
    base = wid * NCHUNK
    _zero_acc(acc, zer, s)
    plsc.subcore_barrier()
    rows = (rows0, rows1)
    sems = (sem0, sem1)
    def stage(st, carry):
        sbase = base + st * SCH
        pltpu.sync_copy(src2d.at[pl.ds(sbase, SCH)], sidx)
        pltpu.sync_copy(dst2d.at[pl.ds(sbase, SCH)], didx)
        for b in range(NBUF):
            pltpu.async_copy(g.at[sidx.at[b]], rows[b], sems[b])
        def body(gi, c2):
            for b in range(NBUF):
                i = gi * NBUF + b
                pltpu.make_async_copy(g.at[sidx.at[i]], rows[b], sems[b]).wait()
                pltpu.sync_copy(rows[b], acc.at[didx.at[i]], add=True)
                @pl.when(i + NBUF < SCH)
                def _():
                    pltpu.async_copy(g.at[sidx.at[i + NBUF]], rows[b], sems[b])
            return c2
        lax.fori_loop(0, SCH // NBUF, body, 0)
        return carry
    lax.fori_loop(0, NSTAGE, stage, 0)
    plsc.subcore_barrier()
    _copy_out(acc, o0, o1, c, s)


_segsum_call = pl.kernel(
    _segsum_body,
    out_type=(jax.ShapeDtypeStruct((NP, D), jnp.float32),
              jax.ShapeDtypeStruct((NP, D), jnp.float32)),
    mesh=_mesh,
    scratch_types=[
        pltpu.VMEM((SCH, CHUNK), jnp.int32),
        pltpu.VMEM((SCH, CHUNK), jnp.int32),
        pltpu.VMEM((CHUNK, D), jnp.float32),
        pltpu.VMEM((CHUNK, D), jnp.float32),
        pltpu.VMEM_SHARED((ACC_ROWS, D), jnp.float32),
        pltpu.SemaphoreType.DMA,
        pltpu.SemaphoreType.DMA,
    ],
)


def _gather2_body(p, q, src2d, dst2d, pa, qa, sidx, didx, pr0, pr1,
                  qr0, qr1, psem0, psem1, qsem0, qsem1):
    c = lax.axis_index("c")
    s = lax.axis_index("s")
    wid = s * NC + c
    base = wid * NCHUNK
    prows = (pr0, pr1)
    qrows = (qr0, qr1)
    psems = (psem0, psem1)
    qsems = (qsem0, qsem1)
    def stage(st, carry):
        sbase = base + st * SCH
        pltpu.sync_copy(src2d.at[pl.ds(sbase, SCH)], sidx)
        pltpu.sync_copy(dst2d.at[pl.ds(sbase, SCH)], didx)
        for b in range(NBUF):
            pltpu.async_copy(p.at[sidx.at[b]], prows[b], psems[b])
            pltpu.async_copy(q.at[didx.at[b]], qrows[b], qsems[b])
        def body(gi, c2):
            for b in range(NBUF):
                i = gi * NBUF + b
                off = (sbase + i) * CHUNK
                pltpu.make_async_copy(p.at[sidx.at[i]], prows[b], psems[b]).wait()
                pltpu.sync_copy(prows[b], pa.at[pl.ds(off, CHUNK)])
                pltpu.make_async_copy(q.at[didx.at[i]], qrows[b], qsems[b]).wait()
                pltpu.sync_copy(qrows[b], qa.at[pl.ds(off, CHUNK)])
                @pl.when(i + NBUF < SCH)
                def _():
                    pltpu.async_copy(p.at[sidx.at[i + NBUF]], prows[b], psems[b])
                    pltpu.async_copy(q.at[didx.at[i + NBUF]], qrows[b], qsems[b])
            return c2
        lax.fori_loop(0, SCH // NBUF, body, 0)
        return carry
    lax.fori_loop(0, NSTAGE, stage, 0)


_gather2_call = pl.kernel(
    _gather2_body,
    out_type=(jax.ShapeDtypeStruct((E_PAD, D), jnp.float32),
              jax.ShapeDtypeStruct((E_PAD, D), jnp.float32)),
    mesh=_mesh,
    scratch_types=[
        pltpu.VMEM((SCH, CHUNK), jnp.int32),
        pltpu.VMEM((SCH, CHUNK), jnp.int32),
    ] + [pltpu.VMEM((CHUNK, D), jnp.float32)] * 4
      + [pltpu.SemaphoreType.DMA] * 4,
)


BN = 640
BE = 1000

_w_spec = pl.BlockSpec((D, D), lambda i: (0, 0))
_b_spec = pl.BlockSpec((1, D), lambda i: (0, 0))
_n_spec = pl.BlockSpec((BN, D), lambda i: (i, 0))
_d_spec = pl.BlockSpec((BN, 1), lambda i: (i, 0))
_e_spec = pl.BlockSpec((BE, D), lambda i: (i, 0))


def _dis_kern(d0, d1, o):
    o[...] = lax.rsqrt(d0[:, 0:1] + d1[:, 0:1] + 1.0)


def _pre_kern(x, w, d, o):
    o[...] = jnp.dot(x[...], w[...], preferred_element_type=jnp.float32) * d[...]


def _mid_kern(a0, a1, g, d, b, w, o):
    dis = d[...]
    h = jnp.maximum(dis * (a0[...] + a1[...] + g[...]) + b[...], 0.0)
    o[...] = jnp.dot(h, w[...], preferred_element_type=jnp.float32) * dis


def _final_kern(a0, a1, g, d, b3, wt, wb, bm1, po, qo):
    h3 = d[...] * (a0[...] + a1[...] + g[...]) + b3[...]
    po[...] = jnp.dot(h3, wt[...], preferred_element_type=jnp.float32) + bm1[...]
    qo[...] = jnp.dot(h3, wb[...], preferred_element_type=jnp.float32)


def _edge_kern(pa, qa, w, b, o):
    h = jnp.maximum(pa[...] + qa[...], 0.0)
    o[...] = jnp.dot(h, w[...], preferred_element_type=jnp.float32) + b[...]


_dis_call = pl.pallas_call(
    _dis_kern,
    grid=(NP // BN,),
    in_specs=[_n_spec, _n_spec],
    out_specs=_d_spec,
    out_shape=jax.ShapeDtypeStruct((NP, 1), jnp.float32),
)

_pre_call = pl.pallas_call(
    _pre_kern,
    grid=(NP // BN,),
    in_specs=[_n_spec, _w_spec, _d_spec],
    out_specs=_n_spec,
    out_shape=jax.ShapeDtypeStruct((NP, D), jnp.float32),
)

_mid_call = pl.pallas_call(
    _mid_kern,
    grid=(NP // BN,),
    in_specs=[_n_spec, _n_spec, _n_spec, _d_spec, _b_spec, _w_spec],
    out_specs=_n_spec,
    out_shape=jax.ShapeDtypeStruct((NP, D), jnp.float32),
)

_final_call = pl.pallas_call(
    _final_kern,
    grid=(NP // BN,),
    in_specs=[_n_spec, _n_spec, _n_spec, _d_spec, _b_spec, _w_spec, _w_spec,
              _b_spec],
    out_specs=(_n_spec, _n_spec),
    out_shape=(jax.ShapeDtypeStruct((NP, D), jnp.float32),
               jax.ShapeDtypeStruct((NP, D), jnp.float32)),
)

_edge_call = pl.pallas_call(
    _edge_kern,
    grid=(E // BE,),
    in_specs=[_e_spec, _e_spec, _w_spec, _b_spec],
    out_specs=_e_spec,
    out_shape=jax.ShapeDtypeStruct((E, D), jnp.float32),
)


def kernel(x, edge_index, W1, b1, W2, b2, W3, b3, Wm1, bm1, Wm2, bm2):
    pad = E_PAD - E
    src2d = jnp.concatenate(
        [edge_index[0], jnp.zeros((pad,), jnp.int32)]).reshape(-1, CHUNK)
    dst2d = jnp.concatenate(
        [edge_index[1], jnp.full((pad,), N, jnp.int32)]).reshape(-1, CHUNK)
    b1r, b2r, b3r = b1.reshape(1, D), b2.reshape(1, D), b3.reshape(1, D)
    bm1r, bm2r = bm1.reshape(1, D), bm2.reshape(1, D)
    wt, wb = Wm1[:D], Wm1[D:]
    xp = jnp.concatenate([x, jnp.zeros((NP - N, D), jnp.float32)])

    zer = jnp.zeros((ZR, D), jnp.float32)
    d0, d1 = _deg_call(dst2d, jnp.ones((CHUNK, D), jnp.float32), zer)
    dis = _dis_call(d0, d1)
    g1 = _pre_call(xp, W1, dis)
    a0, a1 = _segsum_call(g1, src2d, dst2d, zer)
    g2 = _mid_call(a0, a1, g1, dis, b1r, W2)
    a0, a1 = _segsum_call(g2, src2d, dst2d, zer)
    g3 = _mid_call(a0, a1, g2, dis, b2r, W3)
    a0, a1 = _segsum_call(g3, src2d, dst2d, zer)
    p, q = _final_call(a0, a1, g3, dis, b3r, wt, wb, bm1r)
    pa, qa = _gather2_call(p, q, src2d, dst2d)
    return _edge_call(pa, qa, Wm2, bm2r)
